# Initial kernel scaffold; baseline (speedup 1.0000x reference)
import jax
import jax.numpy as jnp
from jax.experimental import pallas as pl


def kernel(lengths, table):
    raise NotImplementedError("write your pallas kernel here")


# speedup vs baseline: 1.7359x; 1.7359x over previous
import functools

import jax
import jax.numpy as jnp
from jax import lax
from jax.experimental import pallas as pl
from jax.experimental.pallas import tpu as pltpu
from jax.experimental.pallas import tpu_sc as plsc

_BINS = (1, 2, 3, 4, 8, 16, 32, 64)
_NUM_EMB = 9
_DIM = 20
_N = 16384

_NC = 2
_NS = 16
_NW = _NC * _NS
_BPW = _N // _NW
_L = 16
_TAB_PAD = 256


def _sc_body(len_hbm, tab_hbm, out_hbm, len_v, tab_v, out_v):
    wid = lax.axis_index("s") * _NC + lax.axis_index("c")
    base = wid * _BPW

    pltpu.sync_copy(len_hbm.at[pl.ds(base, _BPW)], len_v)
    pltpu.sync_copy(tab_hbm, tab_v)

    lane = lax.iota(jnp.int32, _L)

    def body(g, carry):
        lens = len_v[pl.ds(g * _L, _L)]
        acc = jnp.zeros((_L,), jnp.int32)
        for b in _BINS:
            acc = acc + jnp.where(lens > b, 1, 0).astype(jnp.int32)
        pos = acc * _DIM
        dst = (g * _L + lane) * _DIM
        for d in range(_DIM):
            vals = plsc.load_gather(tab_v, [pos + d])
            plsc.store_scatter(out_v, [dst + d], vals)
        return carry

    lax.fori_loop(0, _BPW // _L, body, 0)

    pltpu.sync_copy(out_v, out_hbm.at[pl.ds(base * _DIM, _BPW * _DIM)])


@functools.partial(
    pl.kernel,
    out_type=jax.ShapeDtypeStruct((_N * _DIM,), jnp.float32),
    mesh=plsc.VectorSubcoreMesh(core_axis_name="c", subcore_axis_name="s"),
    compiler_params=pltpu.CompilerParams(needs_layout_passes=False),
    scratch_types=[
        pltpu.VMEM((_BPW,), jnp.int32),
        pltpu.VMEM((_TAB_PAD,), jnp.float32),
        pltpu.VMEM((_BPW * _DIM,), jnp.float32),
    ],
)
def _sc_lookup(len_hbm, tab_hbm, out_hbm, len_v, tab_v, out_v):
    _sc_body(len_hbm, tab_hbm, out_hbm, len_v, tab_v, out_v)


def kernel(lengths, table):
    lengths = lengths.astype(jnp.int32)
    tab = jnp.pad(table.reshape(-1), (0, _TAB_PAD - _NUM_EMB * _DIM))
    out = _sc_lookup(lengths, tab)
    return out.reshape(_N, _DIM)
